# Initial kernel scaffold; baseline (speedup 1.0000x reference)
import functools

import jax
import jax.numpy as jnp
from jax import lax
from jax.experimental import pallas as pl
from jax.experimental.pallas import tpu as pltpu
from jax.experimental.pallas import tpu_sc as plsc

N = 50000
M = 16
F_ATOM = 64
F_NBR = 41
F_OUT = 128
EDGES = N * M
EPS = 1e-5

_B = 400
_C = 1000


def _sigmoid(x):
    return 1.0 / (1.0 + jnp.exp(-x))


def _softplus(x):
    return jnp.maximum(x, 0.0) + jnp.log(1.0 + jnp.exp(-jnp.abs(x)))


def _sc_gather(atom_fea, idx_flat):
    info = plsc.get_sparse_core_info()
    nc, ns = info.num_cores, info.num_subcores
    nw = nc * ns
    bpw = EDGES // nw
    nchunk = bpw // _C
    mesh = plsc.VectorSubcoreMesh(core_axis_name="c", subcore_axis_name="s")

    @functools.partial(
        pl.kernel,
        out_type=jax.ShapeDtypeStruct((EDGES, F_ATOM), jnp.float32),
        mesh=mesh,
        scratch_types=[
            pltpu.VMEM((_C,), jnp.int32),
            pltpu.VMEM((_C, F_ATOM), jnp.float32),
            pltpu.SemaphoreType.DMA,
        ],
    )
    def gather_kernel(table_hbm, idx_hbm, out_hbm, idx_v, rows_v, sem):
        wid = lax.axis_index("s") * nc + lax.axis_index("c")
        base = wid * bpw
        for i in range(nchunk):
            off = base + i * _C
            pltpu.sync_copy(idx_hbm.at[pl.ds(off, _C)], idx_v)
            pltpu.async_copy(table_hbm.at[idx_v], rows_v, sem).wait()
            pltpu.sync_copy(rows_v, out_hbm.at[pl.ds(off, _C)])

    return gather_kernel(atom_fea, idx_flat)


def _edge_gated(gath_ref, nbr_ref, atom_ref, wself_ref, wnbr_ref, wedge_ref,
                bias_ref):
    r = _B * M
    s = jnp.dot(atom_ref[...], wself_ref[...],
                preferred_element_type=jnp.float32) + bias_ref[...]
    gp = jnp.dot(gath_ref[...], wnbr_ref[...],
                 preferred_element_type=jnp.float32)
    e = jnp.dot(nbr_ref[...].reshape(r, F_NBR), wedge_ref[...],
                preferred_element_type=jnp.float32)
    return (gp + e).reshape(_B, M, F_OUT) + s[:, None, :]


def _k1_body(gath_ref, nbr_ref, atom_ref, wself_ref, wnbr_ref, wedge_ref,
             bias_ref, s1_ref, s2_ref):
    gated = _edge_gated(gath_ref, nbr_ref, atom_ref, wself_ref, wnbr_ref,
                        wedge_ref, bias_ref)
    g2 = gated.reshape(_B * M, F_OUT)

    @pl.when(pl.program_id(0) == 0)
    def _():
        s1_ref[...] = jnp.zeros_like(s1_ref)
        s2_ref[...] = jnp.zeros_like(s2_ref)

    s1_ref[...] += jnp.sum(g2, axis=0, keepdims=True)
    s2_ref[...] += jnp.sum(g2 * g2, axis=0, keepdims=True)


def _k2_body(gath_ref, nbr_ref, atom_ref, wself_ref, wnbr_ref, wedge_ref,
             bias_ref, s1_ref, s2_ref, g1_ref, b1_ref,
             ns_ref, t1_ref, t2_ref):
    mean = s1_ref[...] / EDGES
    var = s2_ref[...] / EDGES - mean * mean
    scale = g1_ref[...] * lax.rsqrt(var + EPS)
    shift = b1_ref[...] - mean * scale

    gated = _edge_gated(gath_ref, nbr_ref, atom_ref, wself_ref, wnbr_ref,
                        wedge_ref, bias_ref)
    xh = gated * scale.reshape(1, 1, F_OUT) + shift.reshape(1, 1, F_OUT)
    filt = _sigmoid(xh[..., :F_ATOM])
    core = _softplus(xh[..., F_ATOM:])
    ns = jnp.sum(filt * core, axis=1)
    ns_ref[...] = ns

    @pl.when(pl.program_id(0) == 0)
    def _():
        t1_ref[...] = jnp.zeros_like(t1_ref)
        t2_ref[...] = jnp.zeros_like(t2_ref)

    t1_ref[...] += jnp.sum(ns, axis=0, keepdims=True)
    t2_ref[...] += jnp.sum(ns * ns, axis=0, keepdims=True)


def _k3_body(atom_ref, ns_ref, t1_ref, t2_ref, g2_ref, b2_ref, out_ref):
    mean = t1_ref[...] / N
    var = t2_ref[...] / N - mean * mean
    scale = g2_ref[...] * lax.rsqrt(var + EPS)
    shift = b2_ref[...] - mean * scale
    out_ref[...] = _softplus(atom_ref[...] + ns_ref[...] * scale + shift)


def kernel(atom_fea, nbr_fea, nbr_fea_idx, W_full, b_full,
           bn1_gamma, bn1_beta, bn2_gamma, bn2_beta):
    idx_flat = nbr_fea_idx.reshape(-1).astype(jnp.int32)
    gathered = _sc_gather(atom_fea, idx_flat)

    wself = W_full[:, :F_ATOM].T
    wnbr = W_full[:, F_ATOM:2 * F_ATOM].T
    wedge = W_full[:, 2 * F_ATOM:].T
    bias = b_full.reshape(1, F_OUT)
    g1 = bn1_gamma.reshape(1, F_OUT)
    b1 = bn1_beta.reshape(1, F_OUT)
    g2 = bn2_gamma.reshape(1, F_ATOM)
    b2 = bn2_beta.reshape(1, F_ATOM)

    grid = (N // _B,)
    const2 = lambda s: pl.BlockSpec(s, lambda i: (0, 0))
    edge_specs = [
        pl.BlockSpec((_B * M, F_ATOM), lambda i: (i, 0)),
        pl.BlockSpec((_B, M, F_NBR), lambda i: (i, 0, 0)),
        pl.BlockSpec((_B, F_ATOM), lambda i: (i, 0)),
        const2((F_ATOM, F_OUT)),
        const2((F_ATOM, F_OUT)),
        const2((F_NBR, F_OUT)),
        const2((1, F_OUT)),
    ]

    s1, s2 = pl.pallas_call(
        _k1_body,
        grid=grid,
        in_specs=edge_specs,
        out_specs=[const2((1, F_OUT)), const2((1, F_OUT))],
        out_shape=[jax.ShapeDtypeStruct((1, F_OUT), jnp.float32)] * 2,
        compiler_params=pltpu.CompilerParams(
            dimension_semantics=("arbitrary",)),
    )(gathered, nbr_fea, atom_fea, wself, wnbr, wedge, bias)

    ns, t1, t2 = pl.pallas_call(
        _k2_body,
        grid=grid,
        in_specs=edge_specs + [const2((1, F_OUT))] * 4,
        out_specs=[
            pl.BlockSpec((_B, F_ATOM), lambda i: (i, 0)),
            const2((1, F_ATOM)),
            const2((1, F_ATOM)),
        ],
        out_shape=[
            jax.ShapeDtypeStruct((N, F_ATOM), jnp.float32),
            jax.ShapeDtypeStruct((1, F_ATOM), jnp.float32),
            jax.ShapeDtypeStruct((1, F_ATOM), jnp.float32),
        ],
        compiler_params=pltpu.CompilerParams(
            dimension_semantics=("arbitrary",)),
    )(gathered, nbr_fea, atom_fea, wself, wnbr, wedge, bias, s1, s2, g1, b1)

    out = pl.pallas_call(
        _k3_body,
        grid=grid,
        in_specs=[
            pl.BlockSpec((_B, F_ATOM), lambda i: (i, 0)),
            pl.BlockSpec((_B, F_ATOM), lambda i: (i, 0)),
            const2((1, F_ATOM)),
            const2((1, F_ATOM)),
            const2((1, F_ATOM)),
            const2((1, F_ATOM)),
        ],
        out_specs=pl.BlockSpec((_B, F_ATOM), lambda i: (i, 0)),
        out_shape=jax.ShapeDtypeStruct((N, F_ATOM), jnp.float32),
        compiler_params=pltpu.CompilerParams(
            dimension_semantics=("arbitrary",)),
    )(atom_fea, ns, t1, t2, g2, b2)

    return out


# speedup vs baseline: 2.5452x; 2.5452x over previous
import functools

import jax
import jax.numpy as jnp
from jax import lax
from jax.experimental import pallas as pl
from jax.experimental.pallas import tpu as pltpu
from jax.experimental.pallas import tpu_sc as plsc

N = 50000
M = 16
F_ATOM = 64
F_NBR = 41
F_OUT = 128
EDGES = N * M
EPS = 1e-5

_B = 400
_C = 1000


def _sigmoid(x):
    return 1.0 / (1.0 + jnp.exp(-x))


def _softplus(x):
    return jnp.maximum(x, 0.0) + jnp.log(1.0 + jnp.exp(-jnp.abs(x)))


def _g_table_body(atom_ref, wnbr_ref, g_ref):
    g_ref[...] = jnp.dot(atom_ref[...], wnbr_ref[...],
                         preferred_element_type=jnp.float32)


def _g_table(atom_fea, wnbr):
    bp = 2000
    return pl.pallas_call(
        _g_table_body,
        grid=(N // bp,),
        in_specs=[
            pl.BlockSpec((bp, F_ATOM), lambda i: (i, 0)),
            pl.BlockSpec((F_ATOM, F_OUT), lambda i: (0, 0)),
        ],
        out_specs=pl.BlockSpec((bp, F_OUT), lambda i: (i, 0)),
        out_shape=jax.ShapeDtypeStruct((N, F_OUT), jnp.float32),
    )(atom_fea, wnbr)


def _sc_gather(g_table, idx_flat):
    info = plsc.get_sparse_core_info()
    nc, ns = info.num_cores, info.num_subcores
    nw = nc * ns
    bpw = EDGES // nw
    nchunk = bpw // _C
    mesh = plsc.VectorSubcoreMesh(core_axis_name="c", subcore_axis_name="s")

    @functools.partial(
        pl.kernel,
        out_type=jax.ShapeDtypeStruct((EDGES, F_OUT), jnp.float32),
        mesh=mesh,
        scratch_types=[
            pltpu.VMEM((_C,), jnp.int32),
            pltpu.VMEM((_C, F_OUT), jnp.float32),
            pltpu.SemaphoreType.DMA,
        ],
    )
    def gather_kernel(table_hbm, idx_hbm, out_hbm, idx_v, rows_v, sem):
        wid = lax.axis_index("s") * nc + lax.axis_index("c")
        base = wid * bpw
        for i in range(nchunk):
            off = base + i * _C
            pltpu.sync_copy(idx_hbm.at[pl.ds(off, _C)], idx_v)
            pltpu.async_copy(table_hbm.at[idx_v], rows_v, sem).wait()
            pltpu.sync_copy(rows_v, out_hbm.at[pl.ds(off, _C)])

    return gather_kernel(g_table, idx_flat)


def _edge_gated(gath_ref, nbr_ref, atom_ref, wself_ref, wedge_ref, bias_ref):
    r = _B * M
    s = jnp.dot(atom_ref[...], wself_ref[...],
                preferred_element_type=jnp.float32) + bias_ref[...]
    gp = gath_ref[...]
    e = jnp.dot(nbr_ref[...].reshape(r, F_NBR), wedge_ref[...],
                preferred_element_type=jnp.float32)
    return (gp + e).reshape(_B, M, F_OUT) + s[:, None, :]


def _k1_body(gath_ref, nbr_ref, atom_ref, wself_ref, wedge_ref,
             bias_ref, s1_ref, s2_ref):
    gated = _edge_gated(gath_ref, nbr_ref, atom_ref, wself_ref,
                        wedge_ref, bias_ref)
    g2 = gated.reshape(_B * M, F_OUT)

    @pl.when(pl.program_id(0) == 0)
    def _():
        s1_ref[...] = jnp.zeros_like(s1_ref)
        s2_ref[...] = jnp.zeros_like(s2_ref)

    s1_ref[...] += jnp.sum(g2, axis=0, keepdims=True)
    s2_ref[...] += jnp.sum(g2 * g2, axis=0, keepdims=True)


def _k2_body(gath_ref, nbr_ref, atom_ref, wself_ref, wedge_ref,
             bias_ref, s1_ref, s2_ref, g1_ref, b1_ref,
             ns_ref, t1_ref, t2_ref):
    mean = s1_ref[...] / EDGES
    var = s2_ref[...] / EDGES - mean * mean
    scale = g1_ref[...] * lax.rsqrt(var + EPS)
    shift = b1_ref[...] - mean * scale

    gated = _edge_gated(gath_ref, nbr_ref, atom_ref, wself_ref,
                        wedge_ref, bias_ref)
    xh = gated * scale.reshape(1, 1, F_OUT) + shift.reshape(1, 1, F_OUT)
    filt = _sigmoid(xh[..., :F_ATOM])
    core = _softplus(xh[..., F_ATOM:])
    ns = jnp.sum(filt * core, axis=1)
    ns_ref[...] = ns

    @pl.when(pl.program_id(0) == 0)
    def _():
        t1_ref[...] = jnp.zeros_like(t1_ref)
        t2_ref[...] = jnp.zeros_like(t2_ref)

    t1_ref[...] += jnp.sum(ns, axis=0, keepdims=True)
    t2_ref[...] += jnp.sum(ns * ns, axis=0, keepdims=True)


def _k3_body(atom_ref, ns_ref, t1_ref, t2_ref, g2_ref, b2_ref, out_ref):
    mean = t1_ref[...] / N
    var = t2_ref[...] / N - mean * mean
    scale = g2_ref[...] * lax.rsqrt(var + EPS)
    shift = b2_ref[...] - mean * scale
    out_ref[...] = _softplus(atom_ref[...] + ns_ref[...] * scale + shift)


def kernel(atom_fea, nbr_fea, nbr_fea_idx, W_full, b_full,
           bn1_gamma, bn1_beta, bn2_gamma, bn2_beta):
    idx_flat = nbr_fea_idx.reshape(-1).astype(jnp.int32)
    wself = W_full[:, :F_ATOM].T
    wnbr = W_full[:, F_ATOM:2 * F_ATOM].T
    wedge = W_full[:, 2 * F_ATOM:].T
    g_table = _g_table(atom_fea, wnbr)
    gathered = _sc_gather(g_table, idx_flat)
    bias = b_full.reshape(1, F_OUT)
    g1 = bn1_gamma.reshape(1, F_OUT)
    b1 = bn1_beta.reshape(1, F_OUT)
    g2 = bn2_gamma.reshape(1, F_ATOM)
    b2 = bn2_beta.reshape(1, F_ATOM)

    grid = (N // _B,)
    const2 = lambda s: pl.BlockSpec(s, lambda i: (0, 0))
    edge_specs = [
        pl.BlockSpec((_B * M, F_OUT), lambda i: (i, 0)),
        pl.BlockSpec((_B, M, F_NBR), lambda i: (i, 0, 0)),
        pl.BlockSpec((_B, F_ATOM), lambda i: (i, 0)),
        const2((F_ATOM, F_OUT)),
        const2((F_NBR, F_OUT)),
        const2((1, F_OUT)),
    ]

    s1, s2 = pl.pallas_call(
        _k1_body,
        grid=grid,
        in_specs=edge_specs,
        out_specs=[const2((1, F_OUT)), const2((1, F_OUT))],
        out_shape=[jax.ShapeDtypeStruct((1, F_OUT), jnp.float32)] * 2,
        compiler_params=pltpu.CompilerParams(
            dimension_semantics=("arbitrary",)),
    )(gathered, nbr_fea, atom_fea, wself, wedge, bias)

    ns, t1, t2 = pl.pallas_call(
        _k2_body,
        grid=grid,
        in_specs=edge_specs + [const2((1, F_OUT))] * 4,
        out_specs=[
            pl.BlockSpec((_B, F_ATOM), lambda i: (i, 0)),
            const2((1, F_ATOM)),
            const2((1, F_ATOM)),
        ],
        out_shape=[
            jax.ShapeDtypeStruct((N, F_ATOM), jnp.float32),
            jax.ShapeDtypeStruct((1, F_ATOM), jnp.float32),
            jax.ShapeDtypeStruct((1, F_ATOM), jnp.float32),
        ],
        compiler_params=pltpu.CompilerParams(
            dimension_semantics=("arbitrary",)),
    )(gathered, nbr_fea, atom_fea, wself, wedge, bias, s1, s2, g1, b1)

    out = pl.pallas_call(
        _k3_body,
        grid=grid,
        in_specs=[
            pl.BlockSpec((_B, F_ATOM), lambda i: (i, 0)),
            pl.BlockSpec((_B, F_ATOM), lambda i: (i, 0)),
            const2((1, F_ATOM)),
            const2((1, F_ATOM)),
            const2((1, F_ATOM)),
            const2((1, F_ATOM)),
        ],
        out_specs=pl.BlockSpec((_B, F_ATOM), lambda i: (i, 0)),
        out_shape=jax.ShapeDtypeStruct((N, F_ATOM), jnp.float32),
        compiler_params=pltpu.CompilerParams(
            dimension_semantics=("arbitrary",)),
    )(atom_fea, ns, t1, t2, g2, b2)

    return out


# speedup vs baseline: 2.6689x; 1.0486x over previous
import functools

import jax
import jax.numpy as jnp
from jax import lax
from jax.experimental import pallas as pl
from jax.experimental.pallas import tpu as pltpu
from jax.experimental.pallas import tpu_sc as plsc

N = 50000
M = 16
F_ATOM = 64
F_NBR = 41
F_OUT = 128
EDGES = N * M
EPS = 1e-5

_B = 400
_C = 1000


def _sigmoid(x):
    return 1.0 / (1.0 + jnp.exp(-x))


def _softplus(x):
    return jnp.maximum(x, 0.0) + jnp.log(1.0 + jnp.exp(-jnp.abs(x)))


def _g_table_body(atom_ref, wnbr_ref, g_ref):
    g_ref[...] = jnp.dot(atom_ref[...], wnbr_ref[...],
                         preferred_element_type=jnp.float32)


def _g_table(atom_fea, wnbr):
    bp = 2000
    return pl.pallas_call(
        _g_table_body,
        grid=(N // bp,),
        in_specs=[
            pl.BlockSpec((bp, F_ATOM), lambda i: (i, 0)),
            pl.BlockSpec((F_ATOM, F_OUT), lambda i: (0, 0)),
        ],
        out_specs=pl.BlockSpec((bp, F_OUT), lambda i: (i, 0)),
        out_shape=jax.ShapeDtypeStruct((N, F_OUT), jnp.float32),
    )(atom_fea, wnbr)


def _sc_gather(g_table, idx_flat):
    info = plsc.get_sparse_core_info()
    nc, ns = info.num_cores, info.num_subcores
    nw = nc * ns
    bpw = EDGES // nw
    nchunk = bpw // _C
    mesh = plsc.VectorSubcoreMesh(core_axis_name="c", subcore_axis_name="s")

    @functools.partial(
        pl.kernel,
        out_type=jax.ShapeDtypeStruct((EDGES, F_OUT), jnp.float32),
        mesh=mesh,
        scratch_types=[
            pltpu.VMEM((_C,), jnp.int32),
            pltpu.VMEM((_C, F_OUT), jnp.float32),
            pltpu.SemaphoreType.DMA,
        ],
    )
    def gather_kernel(table_hbm, idx_hbm, out_hbm, idx_v, rows_v, sem):
        wid = lax.axis_index("s") * nc + lax.axis_index("c")
        base = wid * bpw
        for i in range(nchunk):
            off = base + i * _C
            pltpu.sync_copy(idx_hbm.at[pl.ds(off, _C)], idx_v)
            pltpu.async_copy(table_hbm.at[idx_v], rows_v, sem).wait()
            pltpu.sync_copy(rows_v, out_hbm.at[pl.ds(off, _C)])

    return gather_kernel(g_table, idx_flat)


def _edge_gated(gath_ref, nbr_ref, atom_ref, wself_ref, wedge_ref, bias_ref):
    r = _B * M
    s = jnp.dot(atom_ref[...], wself_ref[...],
                preferred_element_type=jnp.float32) + bias_ref[...]
    gp = gath_ref[...]
    e = jnp.dot(nbr_ref[...].reshape(r, F_NBR), wedge_ref[...],
                preferred_element_type=jnp.float32)
    return (gp + e).reshape(_B, M, F_OUT) + s[:, None, :]


def _k1_body(gath_ref, nbr_ref, atom_ref, wself_ref, wedge_ref,
             bias_ref, s1_ref, s2_ref, xh_ref):
    gated = _edge_gated(gath_ref, nbr_ref, atom_ref, wself_ref,
                        wedge_ref, bias_ref)
    g2 = gated.reshape(_B * M, F_OUT)
    xh_ref[...] = g2.astype(jnp.bfloat16)

    @pl.when(pl.program_id(0) == 0)
    def _():
        s1_ref[...] = jnp.zeros_like(s1_ref)
        s2_ref[...] = jnp.zeros_like(s2_ref)

    s1_ref[...] += jnp.sum(g2, axis=0, keepdims=True)
    s2_ref[...] += jnp.sum(g2 * g2, axis=0, keepdims=True)


def _k2_body(xh_in_ref, s1_ref, s2_ref, g1_ref, b1_ref,
             ns_ref, t1_ref, t2_ref):
    mean = s1_ref[...] / EDGES
    var = s2_ref[...] / EDGES - mean * mean
    scale = g1_ref[...] * lax.rsqrt(var + EPS)
    shift = b1_ref[...] - mean * scale

    gated = xh_in_ref[...].astype(jnp.float32).reshape(_B, M, F_OUT)
    xh = gated * scale.reshape(1, 1, F_OUT) + shift.reshape(1, 1, F_OUT)
    filt = _sigmoid(xh[..., :F_ATOM])
    core = _softplus(xh[..., F_ATOM:])
    ns = jnp.sum(filt * core, axis=1)
    ns_ref[...] = ns

    @pl.when(pl.program_id(0) == 0)
    def _():
        t1_ref[...] = jnp.zeros_like(t1_ref)
        t2_ref[...] = jnp.zeros_like(t2_ref)

    t1_ref[...] += jnp.sum(ns, axis=0, keepdims=True)
    t2_ref[...] += jnp.sum(ns * ns, axis=0, keepdims=True)


def _k3_body(atom_ref, ns_ref, t1_ref, t2_ref, g2_ref, b2_ref, out_ref):
    mean = t1_ref[...] / N
    var = t2_ref[...] / N - mean * mean
    scale = g2_ref[...] * lax.rsqrt(var + EPS)
    shift = b2_ref[...] - mean * scale
    out_ref[...] = _softplus(atom_ref[...] + ns_ref[...] * scale + shift)


def kernel(atom_fea, nbr_fea, nbr_fea_idx, W_full, b_full,
           bn1_gamma, bn1_beta, bn2_gamma, bn2_beta):
    idx_flat = nbr_fea_idx.reshape(-1).astype(jnp.int32)
    wself = W_full[:, :F_ATOM].T
    wnbr = W_full[:, F_ATOM:2 * F_ATOM].T
    wedge = W_full[:, 2 * F_ATOM:].T
    g_table = _g_table(atom_fea, wnbr)
    gathered = _sc_gather(g_table, idx_flat)
    bias = b_full.reshape(1, F_OUT)
    g1 = bn1_gamma.reshape(1, F_OUT)
    b1 = bn1_beta.reshape(1, F_OUT)
    g2 = bn2_gamma.reshape(1, F_ATOM)
    b2 = bn2_beta.reshape(1, F_ATOM)

    grid = (N // _B,)
    const2 = lambda s: pl.BlockSpec(s, lambda i: (0, 0))
    edge_specs = [
        pl.BlockSpec((_B * M, F_OUT), lambda i: (i, 0)),
        pl.BlockSpec((_B, M, F_NBR), lambda i: (i, 0, 0)),
        pl.BlockSpec((_B, F_ATOM), lambda i: (i, 0)),
        const2((F_ATOM, F_OUT)),
        const2((F_NBR, F_OUT)),
        const2((1, F_OUT)),
    ]

    s1, s2, xh = pl.pallas_call(
        _k1_body,
        grid=grid,
        in_specs=edge_specs,
        out_specs=[const2((1, F_OUT)), const2((1, F_OUT)),
                   pl.BlockSpec((_B * M, F_OUT), lambda i: (i, 0))],
        out_shape=[jax.ShapeDtypeStruct((1, F_OUT), jnp.float32)] * 2
        + [jax.ShapeDtypeStruct((EDGES, F_OUT), jnp.bfloat16)],
        compiler_params=pltpu.CompilerParams(
            dimension_semantics=("arbitrary",)),
    )(gathered, nbr_fea, atom_fea, wself, wedge, bias)

    ns, t1, t2 = pl.pallas_call(
        _k2_body,
        grid=grid,
        in_specs=[pl.BlockSpec((_B * M, F_OUT), lambda i: (i, 0))]
        + [const2((1, F_OUT))] * 4,
        out_specs=[
            pl.BlockSpec((_B, F_ATOM), lambda i: (i, 0)),
            const2((1, F_ATOM)),
            const2((1, F_ATOM)),
        ],
        out_shape=[
            jax.ShapeDtypeStruct((N, F_ATOM), jnp.float32),
            jax.ShapeDtypeStruct((1, F_ATOM), jnp.float32),
            jax.ShapeDtypeStruct((1, F_ATOM), jnp.float32),
        ],
        compiler_params=pltpu.CompilerParams(
            dimension_semantics=("arbitrary",)),
    )(xh, s1, s2, g1, b1)

    b3 = 2000
    out = pl.pallas_call(
        _k3_body,
        grid=(N // b3,),
        in_specs=[
            pl.BlockSpec((b3, F_ATOM), lambda i: (i, 0)),
            pl.BlockSpec((b3, F_ATOM), lambda i: (i, 0)),
            const2((1, F_ATOM)),
            const2((1, F_ATOM)),
            const2((1, F_ATOM)),
            const2((1, F_ATOM)),
        ],
        out_specs=pl.BlockSpec((b3, F_ATOM), lambda i: (i, 0)),
        out_shape=jax.ShapeDtypeStruct((N, F_ATOM), jnp.float32),
        compiler_params=pltpu.CompilerParams(
            dimension_semantics=("parallel",)),
    )(atom_fea, ns, t1, t2, g2, b2)

    return out
